# baseline (speedup 1.0000x reference)
import functools

import jax
import jax.numpy as jnp
from jax import lax
from jax.experimental import pallas as pl
from jax.experimental.pallas import tpu as pltpu
from jax.experimental.pallas import tpu_sc as plsc

EMBED = 32
BATCH = 16384
NC, NS, L = 2, 16, 16
NW = NC * NS
BPW = BATCH // NW
CH = 128
NCH = BPW // CH

_mesh = plsc.VectorSubcoreMesh(core_axis_name="c", subcore_axis_name="s")


@functools.partial(
    pl.kernel,
    out_type=jax.ShapeDtypeStruct((BATCH,), jnp.float32),
    mesh=_mesh,
    compiler_params=pltpu.CompilerParams(
        use_tc_tiling_on_sc=False, needs_layout_passes=False),
    scratch_types=[
        pltpu.VMEM((NCH, CH), jnp.int32),
        pltpu.VMEM((NCH, CH), jnp.int32),
        pltpu.VMEM((EMBED, BPW), jnp.float32),
        pltpu.VMEM((EMBED, BPW), jnp.float32),
        pltpu.VMEM((BPW,), jnp.float32),
        pltpu.SemaphoreType.DMA,
    ],
)
def _sc_dots(x_hbm, y_hbm, ct_hbm, ot_hbm, dots_hbm, xv, yv, cbuf, obuf, dv,
             sem):
    wid = lax.axis_index("s") * NC + lax.axis_index("c")
    base = wid * BPW
    pltpu.sync_copy(x_hbm.at[pl.ds(wid * NCH, NCH)], xv)
    pltpu.sync_copy(y_hbm.at[pl.ds(wid * NCH, NCH)], yv)
    for j in range(NCH):
        copies = []
        for c in range(EMBED):
            copies.append(
                pltpu.async_copy(ct_hbm.at[c].at[xv.at[j]],
                                 cbuf.at[c, pl.ds(j * CH, CH)], sem))
            copies.append(
                pltpu.async_copy(ot_hbm.at[c].at[yv.at[j]],
                                 obuf.at[c, pl.ds(j * CH, CH)], sem))
        for cp in copies:
            cp.wait()

    def body(g, carry):
        sl = pl.ds(g * L, L)
        acc = cbuf[0, sl] * obuf[0, sl]
        for c in range(1, EMBED):
            acc = acc + cbuf[c, sl] * obuf[c, sl]
        dv[sl] = acc
        return carry

    lax.fori_loop(0, BPW // L, body, 0)
    pltpu.sync_copy(dv, dots_hbm.at[pl.ds(base, BPW)])


def _tc_loss_body(d_ref, o_ref):
    d = d_ref[...]
    neg_abs = -jnp.abs(d)
    ls = jnp.minimum(d, 0.0) - jnp.log(1.0 + jnp.exp(neg_abs))
    o_ref[0, 0] = -jnp.sum(ls) / BATCH


_tc_loss = pl.pallas_call(
    _tc_loss_body,
    out_shape=jax.ShapeDtypeStruct((1, 1), jnp.float32),
    out_specs=pl.BlockSpec(memory_space=pltpu.SMEM),
)


def kernel(x, y, center_weight, out_weight):
    ct = center_weight.T
    ot = out_weight.T
    x2 = x.reshape(NW * NCH, CH)
    y2 = y.reshape(NW * NCH, CH)
    dots = _sc_dots(x2, y2, ct, ot)
    loss = _tc_loss(dots.reshape(BATCH // 128, 128))
    return loss[0, 0]


# speedup vs baseline: 5.6006x; 5.6006x over previous
import functools

import jax
import jax.numpy as jnp
from jax import lax
from jax.experimental import pallas as pl
from jax.experimental.pallas import tpu as pltpu
from jax.experimental.pallas import tpu_sc as plsc

EMBED = 32
BATCH = 16384
GROUP = 128 // EMBED
NC, NS, L = 2, 16, 16
NW = NC * NS
BPW = BATCH // NW
CH = 128
NCH = BPW // CH

_mesh = plsc.VectorSubcoreMesh(core_axis_name="c", subcore_axis_name="s")


@functools.partial(
    pl.kernel,
    out_type=jax.ShapeDtypeStruct((BATCH,), jnp.float32),
    mesh=_mesh,
    compiler_params=pltpu.CompilerParams(
        use_tc_tiling_on_sc=False, needs_layout_passes=False),
    scratch_types=[
        pltpu.VMEM((BPW,), jnp.int32),
        pltpu.VMEM((BPW,), jnp.int32),
        pltpu.VMEM((NCH, CH), jnp.int32),
        pltpu.VMEM((NCH, CH), jnp.int32),
        pltpu.VMEM((BPW,), jnp.int32),
        pltpu.VMEM((BPW,), jnp.int32),
        pltpu.VMEM((CH, 128), jnp.float32),
        pltpu.VMEM((CH, 128), jnp.float32),
        pltpu.VMEM((CH, 128), jnp.float32),
        pltpu.VMEM((CH, 128), jnp.float32),
        pltpu.VMEM((BPW,), jnp.float32),
        pltpu.SemaphoreType.DMA((2,)),
    ],
)
def _sc_dots(x_hbm, y_hbm, cen_hbm, outw_hbm, dots_hbm, xv, yv, xg, yg, xr, yr,
             cb0, cb1, ob0, ob1, dv, sems):
    wid = lax.axis_index("s") * NC + lax.axis_index("c")
    base = wid * BPW
    pltpu.sync_copy(x_hbm.at[pl.ds(base, BPW)], xv)
    pltpu.sync_copy(y_hbm.at[pl.ds(base, BPW)], yv)

    for k in range(BPW // L):
        sl = pl.ds(k * L, L)
        vx = xv[sl]
        vy = yv[sl]
        r0, c0 = k // (CH // L), (k % (CH // L)) * L
        xg[r0, pl.ds(c0, L)] = lax.shift_right_logical(vx, 2)
        yg[r0, pl.ds(c0, L)] = lax.shift_right_logical(vy, 2)
        xr[sl] = lax.shift_left(jnp.bitwise_and(vx, 3), 5)
        yr[sl] = lax.shift_left(jnp.bitwise_and(vy, 3), 5)

    cbufs = (cb0, cb1)
    obufs = (ob0, ob1)

    def fire(j):
        s = sems.at[j % 2]
        pltpu.async_copy(cen_hbm.at[xg.at[j]], cbufs[j % 2], s)
        pltpu.async_copy(outw_hbm.at[yg.at[j]], obufs[j % 2], s)

    def drain(j):
        s = sems.at[j % 2]
        pltpu.make_async_copy(cen_hbm.at[xg.at[j]], cbufs[j % 2], s).wait()
        pltpu.make_async_copy(outw_hbm.at[yg.at[j]], obufs[j % 2], s).wait()

    def compute(j):
        cb = cbufs[j % 2]
        ob = obufs[j % 2]
        lane = lax.iota(jnp.int32, L)

        def body(g, carry):
            rows = g * L + lane
            sl = pl.ds(j * CH + g * L, L)
            colx = xr[sl]
            coly = yr[sl]
            acc = jnp.zeros((L,), jnp.float32)
            for c in range(EMBED):
                a = plsc.load_gather(cb, [rows, colx + c])
                b = plsc.load_gather(ob, [rows, coly + c])
                acc = acc + a * b
            dv[sl] = acc
            return carry

        lax.fori_loop(0, CH // L, body, 0)

    fire(0)
    fire(1)
    for j in range(NCH):
        drain(j)
        compute(j)
        if j + 2 < NCH:
            fire(j + 2)

    pltpu.sync_copy(dv, dots_hbm.at[pl.ds(base, BPW)])


def _tc_loss_body(d_ref, o_ref):
    d = d_ref[...]
    neg_abs = -jnp.abs(d)
    ls = jnp.minimum(d, 0.0) - jnp.log(1.0 + jnp.exp(neg_abs))
    o_ref[0, 0] = -jnp.sum(ls) / BATCH


_tc_loss = pl.pallas_call(
    _tc_loss_body,
    out_shape=jax.ShapeDtypeStruct((1, 1), jnp.float32),
    out_specs=pl.BlockSpec(memory_space=pltpu.SMEM),
)


def kernel(x, y, center_weight, out_weight):
    cw4 = center_weight.reshape(250000, 128)
    ow4 = out_weight.reshape(250000, 128)
    dots = _sc_dots(x, y, cw4, ow4)
    loss = _tc_loss(dots.reshape(BATCH // 128, 128))
    return loss[0, 0]
